# Initial kernel scaffold; baseline (speedup 1.0000x reference)
import jax
import jax.numpy as jnp
from jax.experimental import pallas as pl


def kernel(data, adj_indices, adj_values, W):
    raise NotImplementedError("write your pallas kernel here")


# speedup vs baseline: 2.7491x; 2.7491x over previous
import functools

import jax
import jax.numpy as jnp
from jax import lax
from jax.experimental import pallas as pl
from jax.experimental.pallas import tpu as pltpu
from jax.experimental.pallas import tpu_sc as plsc

N = 10000
E = 320000
D = 128
H = 128
ALPHA = 0.01

NC = 2
NS = 16
NW = NC * NS
K = 128
NCH = 80
CB = 16
NB = NCH // CB
EPT = NCH * K
EPAD = NW * EPT
LANES = 16
RPT = 624
TAIL = N - RPT * NS


def _mm_body(d_ref, n_ref, w_ref, o_ref):
    feat = d_ref[...] + ALPHA * n_ref[...]
    o_ref[...] = jnp.dot(feat, w_ref[...], preferred_element_type=jnp.float32)


def _matmul(data, noise, W):
    blk = 1000
    return pl.pallas_call(
        _mm_body,
        grid=(N // blk,),
        in_specs=[
            pl.BlockSpec((blk, D), lambda i: (i, 0)),
            pl.BlockSpec((blk, D), lambda i: (i, 0)),
            pl.BlockSpec((D, H), lambda i: (0, 0)),
        ],
        out_specs=pl.BlockSpec((blk, H), lambda i: (i, 0)),
        out_shape=jax.ShapeDtypeStruct((N, H), jnp.float32),
    )(data, noise, W)


def _spmm_body(x_hbm, row_hbm, col_hbm, val_hbm, out_hbm,
               rowb, colb, valb, buf0, acc, g0):
    cid = lax.axis_index("c")
    sid = lax.axis_index("s")
    wid = cid * NS + sid

    zeros16 = jnp.zeros((LANES,), jnp.float32)

    def _zrow(e, carry):
        for u in range(H // LANES):
            buf0[e, pl.ds(u * LANES, LANES)] = zeros16
        return carry

    lax.fori_loop(0, K, _zrow, 0)

    base_r = pl.multiple_of(sid * RPT, 8)
    rem = RPT % K
    for q in range(RPT // K):
        pltpu.sync_copy(buf0, acc.at[pl.ds(base_r + q * K, K)])
    if rem:
        pltpu.sync_copy(buf0.at[pl.ds(0, rem)],
                        acc.at[pl.ds(base_r + (RPT // K) * K, rem)])

    @pl.when(sid == NS - 1)
    def _zero_tail():
        pltpu.sync_copy(buf0.at[pl.ds(0, TAIL)], acc.at[pl.ds(RPT * NS, TAIL)])

    plsc.subcore_barrier()

    def _block(b, carry):
        pltpu.sync_copy(row_hbm.at[wid, b], rowb)
        pltpu.sync_copy(col_hbm.at[wid, b], colb)
        pltpu.sync_copy(val_hbm.at[wid, b], valb)

        def _chunk(j, c1):
            pltpu.async_copy(x_hbm.at[colb.at[j]], buf0, g0).wait()

            def _group(g, c2):
                vv = valb[j, pl.ds(g * LANES, LANES)]
                for e16 in range(LANES):
                    v = vv[e16]
                    e = g * LANES + e16
                    for u in range(H // LANES):
                        sl = pl.ds(u * LANES, LANES)
                        buf0[e, sl] = buf0[e, sl] * v
                return c2

            lax.fori_loop(0, K // LANES, _group, 0)
            pltpu.sync_copy(buf0, acc.at[rowb.at[j]], add=True)
            return c1

        lax.fori_loop(0, CB, _chunk, 0)
        return carry

    lax.fori_loop(0, NB, _block, 0)
    plsc.subcore_barrier()

    for q in range(RPT // K):
        sl = pl.ds(base_r + q * K, K)
        pltpu.sync_copy(acc.at[sl], out_hbm.at[cid].at[sl])
    if rem:
        sl = pl.ds(base_r + (RPT // K) * K, rem)
        pltpu.sync_copy(acc.at[sl], out_hbm.at[cid].at[sl])

    @pl.when(sid == NS - 1)
    def _drain_tail():
        sl = pl.ds(RPT * NS, TAIL)
        pltpu.sync_copy(acc.at[sl], out_hbm.at[cid].at[sl])


@functools.cache
def _make_spmm():
    return pl.kernel(
        _spmm_body,
        out_type=jax.ShapeDtypeStruct((NC, N, H), jnp.float32),
        mesh=plsc.VectorSubcoreMesh(core_axis_name="c", subcore_axis_name="s",
                                    num_cores=NC, num_subcores=NS),
        scratch_types=[
            pltpu.VMEM((CB, K), jnp.int32),
            pltpu.VMEM((CB, K), jnp.int32),
            pltpu.VMEM((CB, K), jnp.float32),
            pltpu.VMEM((K, H), jnp.float32),
            pltpu.VMEM_SHARED((N, H), jnp.float32),
            pltpu.SemaphoreType.DMA,
        ],
    )


def _elu_body(p_ref, o_ref):
    s = p_ref[0] + p_ref[1]
    o_ref[...] = jnp.where(s > 0, s, jnp.exp(jnp.minimum(s, 0.0)) - 1.0)


def _elu_combine(partials):
    blk = 1000
    return pl.pallas_call(
        _elu_body,
        grid=(N // blk,),
        in_specs=[pl.BlockSpec((NC, blk, H), lambda i: (0, i, 0))],
        out_specs=pl.BlockSpec((blk, H), lambda i: (i, 0)),
        out_shape=jax.ShapeDtypeStruct((N, H), jnp.float32),
    )(partials)


def kernel(data, adj_indices, adj_values, W):
    noise = jax.random.normal(jax.random.key(42), (N, D), dtype=jnp.float32)
    x = _matmul(data, noise, W)
    pad = EPAD - E
    row = jnp.pad(adj_indices[0], (0, pad)).reshape(NW, NB, CB, K)
    col = jnp.pad(adj_indices[1], (0, pad)).reshape(NW, NB, CB, K)
    val = jnp.pad(adj_values, (0, pad)).reshape(NW, NB, CB, K)
    partials = _make_spmm()(x, row, col, val)
    return _elu_combine(partials)


# speedup vs baseline: 3.5956x; 1.3079x over previous
import functools

import jax
import jax.numpy as jnp
from jax import lax
from jax.experimental import pallas as pl
from jax.experimental.pallas import tpu as pltpu
from jax.experimental.pallas import tpu_sc as plsc

N = 10000
E = 320000
D = 128
H = 128
ALPHA = 0.01

NC = 2
NS = 16
HH = H // NC
K = 128
NCH = 160
EPT = NCH * K
EPAD = NS * EPT
LANES = 16
RPT = 624
TAIL = N - RPT * NS


def _mm_body(d_ref, n_ref, w_ref, o_ref):
    feat = d_ref[...] + ALPHA * n_ref[...]
    y = jnp.dot(feat, w_ref[...], preferred_element_type=jnp.float32)
    o_ref[0] = y[:, :HH]
    o_ref[1] = y[:, HH:]


def _matmul(data, noise, W):
    blk = 1000
    return pl.pallas_call(
        _mm_body,
        grid=(N // blk,),
        in_specs=[
            pl.BlockSpec((blk, D), lambda i: (i, 0)),
            pl.BlockSpec((blk, D), lambda i: (i, 0)),
            pl.BlockSpec((D, H), lambda i: (0, 0)),
        ],
        out_specs=pl.BlockSpec((NC, blk, HH), lambda i: (0, i, 0)),
        out_shape=jax.ShapeDtypeStruct((NC, N, HH), jnp.float32),
    )(data, noise, W)


def _spmm_body(xs_hbm, row_hbm, col_hbm, val_hbm, out_hbm,
               row_v, col_v, val_v, buf0, buf1, acc, g0, g1):
    cid = lax.axis_index("c")
    sid = lax.axis_index("s")
    x_hbm = xs_hbm.at[cid]

    pltpu.sync_copy(row_hbm.at[sid], row_v)
    pltpu.sync_copy(col_hbm.at[sid], col_v)
    pltpu.sync_copy(val_hbm.at[sid], val_v)

    zeros16 = jnp.zeros((LANES,), jnp.float32)

    def _zrow(e, carry):
        for u in range(HH // LANES):
            buf0[e, pl.ds(u * LANES, LANES)] = zeros16
        return carry

    lax.fori_loop(0, K, _zrow, 0)

    base_r = pl.multiple_of(sid * RPT, 8)
    rem = RPT % K
    for q in range(RPT // K):
        pltpu.sync_copy(buf0, acc.at[pl.ds(base_r + q * K, K)])
    if rem:
        pltpu.sync_copy(buf0.at[pl.ds(0, rem)],
                        acc.at[pl.ds(base_r + (RPT // K) * K, rem)])

    @pl.when(sid == NS - 1)
    def _zero_tail():
        pltpu.sync_copy(buf0.at[pl.ds(0, TAIL)], acc.at[pl.ds(RPT * NS, TAIL)])

    plsc.subcore_barrier()

    def _scale_scatter(j, buf):
        def _group(g, c2):
            vv = val_v[j, pl.ds(g * LANES, LANES)]
            for e16 in range(LANES):
                v = vv[e16]
                e = g * LANES + e16
                for u in range(HH // LANES):
                    sl = pl.ds(u * LANES, LANES)
                    buf[e, sl] = buf[e, sl] * v
            return c2

        lax.fori_loop(0, K // LANES, _group, 0)
        pltpu.sync_copy(buf, acc.at[row_v.at[j]], add=True)

    dummy = x_hbm.at[pl.ds(0, K)]
    pltpu.async_copy(x_hbm.at[col_v.at[0]], buf0, g0)

    def _pair(t, carry):
        j0 = 2 * t
        pltpu.async_copy(x_hbm.at[col_v.at[j0 + 1]], buf1, g1)
        pltpu.make_async_copy(dummy, buf0, g0).wait()
        _scale_scatter(j0, buf0)
        jn = jnp.minimum(j0 + 2, NCH - 1)
        pltpu.async_copy(x_hbm.at[col_v.at[jn]], buf0, g0)
        pltpu.make_async_copy(dummy, buf1, g1).wait()
        _scale_scatter(j0 + 1, buf1)
        return carry

    lax.fori_loop(0, NCH // 2, _pair, 0)
    pltpu.make_async_copy(dummy, buf0, g0).wait()
    plsc.subcore_barrier()

    for q in range(RPT // K):
        sl = pl.ds(base_r + q * K, K)
        pltpu.sync_copy(acc.at[sl], out_hbm.at[cid].at[sl])
    if rem:
        sl = pl.ds(base_r + (RPT // K) * K, rem)
        pltpu.sync_copy(acc.at[sl], out_hbm.at[cid].at[sl])

    @pl.when(sid == NS - 1)
    def _drain_tail():
        sl = pl.ds(RPT * NS, TAIL)
        pltpu.sync_copy(acc.at[sl], out_hbm.at[cid].at[sl])


@functools.cache
def _make_spmm():
    return pl.kernel(
        _spmm_body,
        out_type=jax.ShapeDtypeStruct((NC, N, HH), jnp.float32),
        mesh=plsc.VectorSubcoreMesh(core_axis_name="c", subcore_axis_name="s",
                                    num_cores=NC, num_subcores=NS),
        compiler_params=pltpu.CompilerParams(use_tc_tiling_on_sc=False),
        scratch_types=[
            pltpu.VMEM((NCH, K), jnp.int32),
            pltpu.VMEM((NCH, K), jnp.int32),
            pltpu.VMEM((NCH, K), jnp.float32),
            pltpu.VMEM((K, HH), jnp.float32),
            pltpu.VMEM((K, HH), jnp.float32),
            pltpu.VMEM_SHARED((N, HH), jnp.float32),
            pltpu.SemaphoreType.DMA,
            pltpu.SemaphoreType.DMA,
        ],
    )


def _elu_body(p_ref, o_ref):
    for c in range(NC):
        s = p_ref[c]
        o_ref[:, c * HH:(c + 1) * HH] = jnp.where(
            s > 0, s, jnp.exp(jnp.minimum(s, 0.0)) - 1.0)


def _elu_concat(partials):
    blk = 1000
    return pl.pallas_call(
        _elu_body,
        grid=(N // blk,),
        in_specs=[pl.BlockSpec((NC, blk, HH), lambda i: (0, i, 0))],
        out_specs=pl.BlockSpec((blk, H), lambda i: (i, 0)),
        out_shape=jax.ShapeDtypeStruct((N, H), jnp.float32),
    )(partials)


def kernel(data, adj_indices, adj_values, W):
    noise = jax.random.normal(jax.random.key(42), (N, D), dtype=jnp.float32)
    xs = _matmul(data, noise, W)
    pad = EPAD - E
    row = jnp.pad(adj_indices[0], (0, pad)).reshape(NS, NCH, K)
    col = jnp.pad(adj_indices[1], (0, pad)).reshape(NS, NCH, K)
    val = jnp.pad(adj_values, (0, pad)).reshape(NS, NCH, K)
    partials = _make_spmm()(xs, row, col, val)
    return _elu_concat(partials)


# speedup vs baseline: 4.9784x; 1.3846x over previous
import functools

import jax
import jax.numpy as jnp
from jax import lax
from jax.experimental import pallas as pl
from jax.experimental.pallas import tpu as pltpu
from jax.experimental.pallas import tpu_sc as plsc

N = 10000
E = 320000
D = 128
H = 128
ALPHA = 0.01

NC = 2
NS = 16
HH = H // NC
K = 128
NCH = 160
EPT = NCH * K
EPAD = NS * EPT
LANES = 16
RPT = 624
TAIL = N - RPT * NS


def _mm_body(d_ref, n_ref, w_ref, o_ref):
    feat = d_ref[...] + ALPHA * n_ref[...]
    y = jnp.dot(feat, w_ref[...], preferred_element_type=jnp.float32)
    o_ref[0] = y[:, :HH]
    o_ref[1] = y[:, HH:]


def _matmul(data, noise, W):
    blk = 1000
    return pl.pallas_call(
        _mm_body,
        grid=(N // blk,),
        in_specs=[
            pl.BlockSpec((blk, D), lambda i: (i, 0)),
            pl.BlockSpec((blk, D), lambda i: (i, 0)),
            pl.BlockSpec((D, H), lambda i: (0, 0)),
        ],
        out_specs=pl.BlockSpec((NC, blk, HH), lambda i: (0, i, 0)),
        out_shape=jax.ShapeDtypeStruct((NC, N, HH), jnp.float32),
    )(data, noise, W)


def _spmm_body(xs_hbm, row_hbm, col_hbm, val_hbm, out_hbm,
               row_v, col_v, val_v, buf0, buf1, acc, g0, g1):
    cid = lax.axis_index("c")
    sid = lax.axis_index("s")
    x_hbm = xs_hbm.at[cid]

    pltpu.sync_copy(row_hbm.at[sid], row_v)
    pltpu.sync_copy(col_hbm.at[sid], col_v)
    pltpu.sync_copy(val_hbm.at[sid], val_v)

    zeros16 = jnp.zeros((LANES,), jnp.float32)

    def _zrow(e, carry):
        for u in range(HH // LANES):
            buf0[e, pl.ds(u * LANES, LANES)] = zeros16
        return carry

    lax.fori_loop(0, K, _zrow, 0)

    base_r = pl.multiple_of(sid * RPT, 8)
    rem = RPT % K
    for q in range(RPT // K):
        pltpu.sync_copy(buf0, acc.at[pl.ds(base_r + q * K, K)])
    if rem:
        pltpu.sync_copy(buf0.at[pl.ds(0, rem)],
                        acc.at[pl.ds(base_r + (RPT // K) * K, rem)])

    @pl.when(sid == NS - 1)
    def _zero_tail():
        pltpu.sync_copy(buf0.at[pl.ds(0, TAIL)], acc.at[pl.ds(RPT * NS, TAIL)])

    plsc.subcore_barrier()

    def _scale_scatter(j, buf):
        for g in range(K // LANES):
            vv = val_v[j, pl.ds(g * LANES, LANES)]
            for e16 in range(LANES):
                v = vv[e16]
                e = g * LANES + e16
                for u in range(HH // LANES):
                    sl = pl.ds(u * LANES, LANES)
                    buf[e, sl] = buf[e, sl] * v
        pltpu.sync_copy(buf, acc.at[row_v.at[j]], add=True)

    dummy = x_hbm.at[pl.ds(0, K)]
    pltpu.async_copy(x_hbm.at[col_v.at[0]], buf0, g0)

    def _pair(t, carry):
        j0 = 2 * t
        pltpu.async_copy(x_hbm.at[col_v.at[j0 + 1]], buf1, g1)
        pltpu.make_async_copy(dummy, buf0, g0).wait()
        _scale_scatter(j0, buf0)
        jn = jnp.minimum(j0 + 2, NCH - 1)
        pltpu.async_copy(x_hbm.at[col_v.at[jn]], buf0, g0)
        pltpu.make_async_copy(dummy, buf1, g1).wait()
        _scale_scatter(j0 + 1, buf1)
        return carry

    lax.fori_loop(0, NCH // 2, _pair, 0)
    pltpu.make_async_copy(dummy, buf0, g0).wait()
    plsc.subcore_barrier()

    for q in range(RPT // K):
        sl = pl.ds(base_r + q * K, K)
        pltpu.sync_copy(acc.at[sl], out_hbm.at[cid].at[sl])
    if rem:
        sl = pl.ds(base_r + (RPT // K) * K, rem)
        pltpu.sync_copy(acc.at[sl], out_hbm.at[cid].at[sl])

    @pl.when(sid == NS - 1)
    def _drain_tail():
        sl = pl.ds(RPT * NS, TAIL)
        pltpu.sync_copy(acc.at[sl], out_hbm.at[cid].at[sl])


@functools.cache
def _make_spmm():
    return pl.kernel(
        _spmm_body,
        out_type=jax.ShapeDtypeStruct((NC, N, HH), jnp.float32),
        mesh=plsc.VectorSubcoreMesh(core_axis_name="c", subcore_axis_name="s",
                                    num_cores=NC, num_subcores=NS),
        compiler_params=pltpu.CompilerParams(use_tc_tiling_on_sc=False),
        scratch_types=[
            pltpu.VMEM((NCH, K), jnp.int32),
            pltpu.VMEM((NCH, K), jnp.int32),
            pltpu.VMEM((NCH, K), jnp.float32),
            pltpu.VMEM((K, HH), jnp.float32),
            pltpu.VMEM((K, HH), jnp.float32),
            pltpu.VMEM_SHARED((N, HH), jnp.float32),
            pltpu.SemaphoreType.DMA,
            pltpu.SemaphoreType.DMA,
        ],
    )


def _elu_body(p_ref, o_ref):
    for c in range(NC):
        s = p_ref[c]
        o_ref[:, c * HH:(c + 1) * HH] = jnp.where(
            s > 0, s, jnp.exp(jnp.minimum(s, 0.0)) - 1.0)


def _elu_concat(partials):
    blk = 1000
    return pl.pallas_call(
        _elu_body,
        grid=(N // blk,),
        in_specs=[pl.BlockSpec((NC, blk, HH), lambda i: (0, i, 0))],
        out_specs=pl.BlockSpec((blk, H), lambda i: (i, 0)),
        out_shape=jax.ShapeDtypeStruct((N, H), jnp.float32),
    )(partials)


def kernel(data, adj_indices, adj_values, W):
    noise = jax.random.normal(jax.random.key(42), (N, D), dtype=jnp.float32)
    xs = _matmul(data, noise, W)
    pad = EPAD - E
    row = jnp.pad(adj_indices[0], (0, pad)).reshape(NS, NCH, K)
    col = jnp.pad(adj_indices[1], (0, pad)).reshape(NS, NCH, K)
    val = jnp.pad(adj_values, (0, pad)).reshape(NS, NCH, K)
    partials = _make_spmm()(xs, row, col, val)
    return _elu_concat(partials)
